# baseline (speedup 1.0000x reference)
import functools

import jax
import jax.numpy as jnp
from jax.experimental import pallas as pl
from jax.experimental.pallas import tpu as pltpu

SUB = 8
WA = 8192
WB = 2048
SUBW = 256


def _rotl(x, d):
    return (x << jnp.uint32(d)) | (x >> jnp.uint32(32 - d))


def _threefry_bits(j):
    ks0 = jnp.uint32(0)
    ks1 = jnp.uint32(1)
    ks2 = jnp.uint32(0x1BD11BDA) ^ ks0 ^ ks1
    ks = (ks0, ks1, ks2)
    rotations = ((13, 15, 26, 6), (17, 29, 16, 24))
    x0 = jnp.zeros_like(j) + ks0
    x1 = j + ks1
    for i in range(5):
        for r in rotations[i % 2]:
            x0 = x0 + x1
            x1 = _rotl(x1, r)
            x1 = x1 ^ x0
        x0 = x0 + ks[(i + 1) % 3]
        x1 = x1 + ks[(i + 2) % 3] + jnp.uint32(i + 1)
    return x0 ^ x1


def _stats_body(temps_ref, logits_ref, m_ref, g_ref, s_ref,
                m_acc, g_acc, s_acc, *, C, nblk):
    r = pl.program_id(0)
    c = pl.program_id(1)

    @pl.when(c == 0)
    def _init():
        m_acc[0] = -jnp.inf
        g_acc[0] = jnp.int32(SUB * C)
        s_acc[0] = jnp.float32(0.0)

    t_raw = temps_ref[r]
    t = jnp.where(t_raw < 1e-5, jnp.float32(1.0), t_raw)
    rt = jnp.float32(1.0) / t
    big = jnp.int32(SUB * C)
    neginf = jnp.float32(-jnp.inf)

    chm = neginf
    chidx = big
    scaled_chunks = []
    valids = []
    flats = []
    for k in range(WA // SUBW):
        x = logits_ref[0, :, k * SUBW:(k + 1) * SUBW]
        scaled = x * rt
        sub = jax.lax.broadcasted_iota(jnp.int32, x.shape, 0)
        lane = jax.lax.broadcasted_iota(jnp.int32, x.shape, 1)
        col = c * WA + k * SUBW + lane
        valid = col < C
        flat = sub * C + col
        sc = jnp.where(valid, scaled, neginf)
        km = jnp.max(sc)
        kidx = jnp.min(jnp.where(sc == km, flat, big))
        chidx = jnp.where(km > chm, kidx, chidx)
        chm = jnp.maximum(chm, km)
        scaled_chunks.append(scaled)
        valids.append(valid)
        flats.append(flat)

    m_old = m_acc[0]
    m_new = jnp.maximum(m_old, chm)
    bsum = jnp.float32(0.0)
    for scaled, valid in zip(scaled_chunks, valids):
        bsum = bsum + jnp.sum(
            jnp.where(valid, jnp.exp(scaled - m_new), jnp.float32(0.0)))
    s_acc[0] = s_acc[0] * jnp.exp(m_old - m_new) + bsum
    g_acc[0] = jnp.where(chm > m_old, chidx, g_acc[0])
    m_acc[0] = m_new

    @pl.when(c == nblk - 1)
    def _emit():
        m_ref[...] = jnp.reshape(m_acc[0], (1, 1, 1))
        g_ref[...] = jnp.reshape(g_acc[0], (1, 1, 1))
        s_ref[...] = jnp.reshape(s_acc[0], (1, 1, 1))


def _sample_body(temps_ref, m_ref, s_ref, g_ref, logits_ref, probs_ref, tok_ref,
                 best_acc, bidx_acc, nan_acc, *, V, C, nblk):
    r = pl.program_id(0)
    c = pl.program_id(1)

    @pl.when(c == 0)
    def _init():
        best_acc[0] = -jnp.inf
        bidx_acc[0] = jnp.int32(0)
        nan_acc[0] = jnp.int32(V)

    t_raw = temps_ref[r]
    t = jnp.where(t_raw < 1e-5, jnp.float32(1.0), t_raw)
    rt = jnp.float32(1.0) / t
    m = m_ref[r]
    rs = jnp.float32(1.0) / s_ref[r]
    big = jnp.int32(V)
    neginf = jnp.float32(-jnp.inf)

    cmx = neginf
    cidx = big
    cnan = big
    for k in range(WB // SUBW):
        x = logits_ref[0, :, k * SUBW:(k + 1) * SUBW]
        e = jnp.exp(x * rt - m)
        probs = e * rs
        probs_ref[0, :, k * SUBW:(k + 1) * SUBW] = probs

        sub = jax.lax.broadcasted_iota(jnp.int32, x.shape, 0)
        lane = jax.lax.broadcasted_iota(jnp.int32, x.shape, 1)
        col = c * WB + k * SUBW + lane
        valid = col < C
        flat = sub * C + col

        j = (r * V + flat).astype(jnp.uint32)
        bits = _threefry_bits(j)
        uf = jax.lax.bitcast_convert_type(
            (bits >> jnp.uint32(9)) | jnp.uint32(0x3F800000), jnp.float32
        ) - jnp.float32(1.0)
        q = -jnp.log1p(-uf)
        ratio = probs / q

        ok = valid & (ratio == ratio)
        r2 = jnp.where(ok, ratio, neginf)
        km = jnp.max(r2)
        kidx = jnp.min(jnp.where(r2 == km, flat, big))
        knan = jnp.min(jnp.where(valid & (ratio != ratio), flat, big))
        cidx = jnp.where(km > cmx, kidx, cidx)
        cmx = jnp.maximum(cmx, km)
        cnan = jnp.minimum(cnan, knan)

    best_old = best_acc[0]
    bidx_acc[0] = jnp.where(cmx > best_old, cidx, bidx_acc[0])
    best_acc[0] = jnp.maximum(best_old, cmx)
    nan_acc[0] = jnp.minimum(nan_acc[0], cnan)

    @pl.when(c == nblk - 1)
    def _emit():
        sampled = jnp.where(nan_acc[0] < big, nan_acc[0], bidx_acc[0])
        tok = jnp.where(t_raw < 1e-5, g_ref[r], sampled)
        tok_ref[...] = jnp.reshape(tok, (1, 1, 1))


def kernel(logits, temperatures):
    B, V = logits.shape
    C = V // SUB
    x3 = logits.reshape(B, SUB, C)

    nblk_a = pl.cdiv(C, WA)
    m3, g3, s3 = pl.pallas_call(
        functools.partial(_stats_body, C=C, nblk=nblk_a),
        grid=(B, nblk_a),
        in_specs=[
            pl.BlockSpec(memory_space=pltpu.SMEM),
            pl.BlockSpec((1, SUB, WA), lambda r, c: (r, 0, c)),
        ],
        out_specs=[
            pl.BlockSpec((1, 1, 1), lambda r, c: (r, 0, 0)),
            pl.BlockSpec((1, 1, 1), lambda r, c: (r, 0, 0)),
            pl.BlockSpec((1, 1, 1), lambda r, c: (r, 0, 0)),
        ],
        out_shape=[
            jax.ShapeDtypeStruct((B, 1, 1), jnp.float32),
            jax.ShapeDtypeStruct((B, 1, 1), jnp.int32),
            jax.ShapeDtypeStruct((B, 1, 1), jnp.float32),
        ],
        scratch_shapes=[
            pltpu.SMEM((1,), jnp.float32),
            pltpu.SMEM((1,), jnp.int32),
            pltpu.SMEM((1,), jnp.float32),
        ],
    )(temperatures, x3)

    nblk_b = pl.cdiv(C, WB)
    probs3, tok3 = pl.pallas_call(
        functools.partial(_sample_body, V=V, C=C, nblk=nblk_b),
        grid=(B, nblk_b),
        in_specs=[
            pl.BlockSpec(memory_space=pltpu.SMEM),
            pl.BlockSpec(memory_space=pltpu.SMEM),
            pl.BlockSpec(memory_space=pltpu.SMEM),
            pl.BlockSpec(memory_space=pltpu.SMEM),
            pl.BlockSpec((1, SUB, WB), lambda r, c: (r, 0, c)),
        ],
        out_specs=[
            pl.BlockSpec((1, SUB, WB), lambda r, c: (r, 0, c)),
            pl.BlockSpec((1, 1, 1), lambda r, c: (r, 0, 0)),
        ],
        out_shape=[
            jax.ShapeDtypeStruct((B, SUB, C), jnp.float32),
            jax.ShapeDtypeStruct((B, 1, 1), jnp.int32),
        ],
        scratch_shapes=[
            pltpu.SMEM((1,), jnp.float32),
            pltpu.SMEM((1,), jnp.int32),
            pltpu.SMEM((1,), jnp.int32),
        ],
    )(temperatures, m3.reshape(B), s3.reshape(B), g3.reshape(B), x3)

    return tok3.reshape(B), probs3.reshape(B, V)


# speedup vs baseline: 5.2692x; 5.2692x over previous
import functools

import jax
import jax.numpy as jnp
from jax.experimental import pallas as pl
from jax.experimental.pallas import tpu as pltpu

SUB = 8
WA = 8192
WB = 8192
SUBW = 512


def _rotl(x, d):
    return (x << jnp.uint32(d)) | (x >> jnp.uint32(32 - d))


def _threefry_bits(j):
    ks0 = jnp.uint32(0)
    ks1 = jnp.uint32(1)
    ks2 = jnp.uint32(0x1BD11BDA) ^ ks0 ^ ks1
    ks = (ks0, ks1, ks2)
    rotations = ((13, 15, 26, 6), (17, 29, 16, 24))
    x0 = jnp.zeros_like(j) + ks0
    x1 = j + ks1
    for i in range(5):
        for r in rotations[i % 2]:
            x0 = x0 + x1
            x1 = _rotl(x1, r)
            x1 = x1 ^ x0
        x0 = x0 + ks[(i + 1) % 3]
        x1 = x1 + ks[(i + 2) % 3] + jnp.uint32(i + 1)
    return x0 ^ x1


def _stats_body(temps_ref, logits_ref, m_ref, g_ref, s_ref,
                vm_acc, vg_acc, vs_acc, *, C, nblk):
    r = pl.program_id(0)
    c = pl.program_id(1)
    neginf = jnp.float32(-jnp.inf)
    big = jnp.int32(SUB * C)

    @pl.when(c == 0)
    def _init():
        vm_acc[...] = jnp.full((SUB, SUBW), neginf, jnp.float32)
        vg_acc[...] = jnp.full((SUB, SUBW), big, jnp.int32)
        vs_acc[...] = jnp.zeros((SUB, SUBW), jnp.float32)

    t_raw = temps_ref[r]
    t = jnp.where(t_raw < 1e-5, jnp.float32(1.0), t_raw)
    rt = jnp.float32(1.0) / t

    sub = jax.lax.broadcasted_iota(jnp.int32, (SUB, SUBW), 0)
    lane = jax.lax.broadcasted_iota(jnp.int32, (SUB, SUBW), 1)
    base_flat = sub * C + lane

    vm = vm_acc[...]
    vg = vg_acc[...]
    vs = vs_acc[...]
    for k in range(WA // SUBW):
        x = logits_ref[0, :, k * SUBW:(k + 1) * SUBW]
        scaled = x * rt
        col0 = c * WA + k * SUBW
        valid = (col0 + lane) < C
        flat = base_flat + col0
        sc = jnp.where(valid, scaled, neginf)
        upd = sc > vm
        vm_new = jnp.maximum(vm, sc)
        vg = jnp.where(upd, flat, vg)
        vs = vs * jnp.exp(vm - vm_new) + jnp.where(
            valid, jnp.exp(scaled - vm_new), jnp.float32(0.0))
        vm = vm_new
    vm_acc[...] = vm
    vg_acc[...] = vg
    vs_acc[...] = vs

    @pl.when(c == nblk - 1)
    def _emit():
        m = jnp.max(vm)
        g = jnp.min(jnp.where(vm == m, vg, big))
        s = jnp.sum(vs * jnp.exp(vm - m))
        m_ref[...] = jnp.reshape(m, (1, 1, 1))
        g_ref[...] = jnp.reshape(g, (1, 1, 1))
        s_ref[...] = jnp.reshape(s, (1, 1, 1))


def _sample_body(temps_ref, m_ref, s_ref, g_ref, logits_ref, probs_ref, tok_ref,
                 vb_acc, vi_acc, vn_acc, *, V, C, nblk):
    r = pl.program_id(0)
    c = pl.program_id(1)
    neginf = jnp.float32(-jnp.inf)
    big = jnp.int32(V)

    @pl.when(c == 0)
    def _init():
        vb_acc[...] = jnp.full((SUB, SUBW), neginf, jnp.float32)
        vi_acc[...] = jnp.zeros((SUB, SUBW), jnp.int32)
        vn_acc[...] = jnp.full((SUB, SUBW), big, jnp.int32)

    t_raw = temps_ref[r]
    t = jnp.where(t_raw < 1e-5, jnp.float32(1.0), t_raw)
    rt = jnp.float32(1.0) / t
    m = m_ref[r]
    rs = jnp.float32(1.0) / s_ref[r]

    sub = jax.lax.broadcasted_iota(jnp.int32, (SUB, SUBW), 0)
    lane = jax.lax.broadcasted_iota(jnp.int32, (SUB, SUBW), 1)
    base_flat = sub * C + lane

    vb = vb_acc[...]
    vi = vi_acc[...]
    vn = vn_acc[...]
    for k in range(WB // SUBW):
        x = logits_ref[0, :, k * SUBW:(k + 1) * SUBW]
        e = jnp.exp(x * rt - m)
        probs = e * rs
        probs_ref[0, :, k * SUBW:(k + 1) * SUBW] = probs

        col0 = c * WB + k * SUBW
        valid = (col0 + lane) < C
        flat = base_flat + col0

        j = (r * V + flat).astype(jnp.uint32)
        bits = _threefry_bits(j)
        uf = jax.lax.bitcast_convert_type(
            (bits >> jnp.uint32(9)) | jnp.uint32(0x3F800000), jnp.float32
        ) - jnp.float32(1.0)
        q = -jnp.log1p(-uf)
        ratio = probs / q

        ok = valid & (ratio == ratio)
        r2 = jnp.where(ok, ratio, neginf)
        upd = r2 > vb
        vb = jnp.maximum(vb, r2)
        vi = jnp.where(upd, flat, vi)
        vn = jnp.minimum(vn, jnp.where(valid & (ratio != ratio), flat, big))
    vb_acc[...] = vb
    vi_acc[...] = vi
    vn_acc[...] = vn

    @pl.when(c == nblk - 1)
    def _emit():
        mx = jnp.max(vb)
        bidx = jnp.min(jnp.where(vb == mx, vi, big))
        nidx = jnp.min(vn)
        sampled = jnp.where(nidx < big, nidx, bidx)
        tok = jnp.where(t_raw < 1e-5, g_ref[r], sampled)
        tok_ref[...] = jnp.reshape(tok, (1, 1, 1))


def kernel(logits, temperatures):
    B, V = logits.shape
    C = V // SUB
    x3 = logits.reshape(B, SUB, C)

    nblk_a = pl.cdiv(C, WA)
    m3, g3, s3 = pl.pallas_call(
        functools.partial(_stats_body, C=C, nblk=nblk_a),
        grid=(B, nblk_a),
        in_specs=[
            pl.BlockSpec(memory_space=pltpu.SMEM),
            pl.BlockSpec((1, SUB, WA), lambda r, c: (r, 0, c)),
        ],
        out_specs=[
            pl.BlockSpec((1, 1, 1), lambda r, c: (r, 0, 0)),
            pl.BlockSpec((1, 1, 1), lambda r, c: (r, 0, 0)),
            pl.BlockSpec((1, 1, 1), lambda r, c: (r, 0, 0)),
        ],
        out_shape=[
            jax.ShapeDtypeStruct((B, 1, 1), jnp.float32),
            jax.ShapeDtypeStruct((B, 1, 1), jnp.int32),
            jax.ShapeDtypeStruct((B, 1, 1), jnp.float32),
        ],
        scratch_shapes=[
            pltpu.VMEM((SUB, SUBW), jnp.float32),
            pltpu.VMEM((SUB, SUBW), jnp.int32),
            pltpu.VMEM((SUB, SUBW), jnp.float32),
        ],
    )(temperatures, x3)

    nblk_b = pl.cdiv(C, WB)
    probs3, tok3 = pl.pallas_call(
        functools.partial(_sample_body, V=V, C=C, nblk=nblk_b),
        grid=(B, nblk_b),
        in_specs=[
            pl.BlockSpec(memory_space=pltpu.SMEM),
            pl.BlockSpec(memory_space=pltpu.SMEM),
            pl.BlockSpec(memory_space=pltpu.SMEM),
            pl.BlockSpec(memory_space=pltpu.SMEM),
            pl.BlockSpec((1, SUB, WB), lambda r, c: (r, 0, c)),
        ],
        out_specs=[
            pl.BlockSpec((1, SUB, WB), lambda r, c: (r, 0, c)),
            pl.BlockSpec((1, 1, 1), lambda r, c: (r, 0, 0)),
        ],
        out_shape=[
            jax.ShapeDtypeStruct((B, SUB, C), jnp.float32),
            jax.ShapeDtypeStruct((B, 1, 1), jnp.int32),
        ],
        scratch_shapes=[
            pltpu.VMEM((SUB, SUBW), jnp.float32),
            pltpu.VMEM((SUB, SUBW), jnp.int32),
            pltpu.VMEM((SUB, SUBW), jnp.int32),
        ],
    )(temperatures, m3.reshape(B), s3.reshape(B), g3.reshape(B), x3)

    return tok3.reshape(B), probs3.reshape(B, V)


# speedup vs baseline: 5.2983x; 1.0055x over previous
import functools

import jax
import jax.numpy as jnp
from jax.experimental import pallas as pl
from jax.experimental.pallas import tpu as pltpu

SUB = 8
WA = 8192
WB = 8192
SUBW = 1024


def _rotl(x, d):
    return (x << jnp.uint32(d)) | (x >> jnp.uint32(32 - d))


def _threefry_bits(j):
    ks0 = jnp.uint32(0)
    ks1 = jnp.uint32(1)
    ks2 = jnp.uint32(0x1BD11BDA) ^ ks0 ^ ks1
    ks = (ks0, ks1, ks2)
    rotations = ((13, 15, 26, 6), (17, 29, 16, 24))
    x0 = jnp.zeros_like(j) + ks0
    x1 = j + ks1
    for i in range(5):
        for r in rotations[i % 2]:
            x0 = x0 + x1
            x1 = _rotl(x1, r)
            x1 = x1 ^ x0
        x0 = x0 + ks[(i + 1) % 3]
        x1 = x1 + ks[(i + 2) % 3] + jnp.uint32(i + 1)
    return x0 ^ x1


def _stats_body(temps_ref, logits_ref, m_ref, g_ref, s_ref,
                vm_acc, vg_acc, vs_acc, *, C, nblk):
    r = pl.program_id(0)
    c = pl.program_id(1)
    neginf = jnp.float32(-jnp.inf)
    big = jnp.int32(SUB * C)

    @pl.when(c == 0)
    def _init():
        vm_acc[...] = jnp.full((SUB, SUBW), neginf, jnp.float32)
        vg_acc[...] = jnp.full((SUB, SUBW), big, jnp.int32)
        vs_acc[...] = jnp.zeros((SUB, SUBW), jnp.float32)

    t_raw = temps_ref[r]
    t = jnp.where(t_raw < 1e-5, jnp.float32(1.0), t_raw)
    rt = jnp.float32(1.0) / t

    sub = jax.lax.broadcasted_iota(jnp.int32, (SUB, SUBW), 0)
    lane = jax.lax.broadcasted_iota(jnp.int32, (SUB, SUBW), 1)
    base_flat = sub * C + lane

    vm = vm_acc[...]
    vg = vg_acc[...]
    vs = vs_acc[...]
    for k in range(WA // SUBW):
        x = logits_ref[0, :, k * SUBW:(k + 1) * SUBW]
        scaled = x * rt
        col0 = c * WA + k * SUBW
        valid = (col0 + lane) < C
        flat = base_flat + col0
        sc = jnp.where(valid, scaled, neginf)
        upd = sc > vm
        vm_new = jnp.maximum(vm, sc)
        vg = jnp.where(upd, flat, vg)
        vs = vs * jnp.exp(vm - vm_new) + jnp.where(
            valid, jnp.exp(scaled - vm_new), jnp.float32(0.0))
        vm = vm_new
    vm_acc[...] = vm
    vg_acc[...] = vg
    vs_acc[...] = vs

    @pl.when(c == nblk - 1)
    def _emit():
        m = jnp.max(vm)
        g = jnp.min(jnp.where(vm == m, vg, big))
        s = jnp.sum(vs * jnp.exp(vm - m))
        m_ref[...] = jnp.reshape(m, (1, 1, 1))
        g_ref[...] = jnp.reshape(g, (1, 1, 1))
        s_ref[...] = jnp.reshape(s, (1, 1, 1))


def _sample_body(temps_ref, m_ref, s_ref, g_ref, logits_ref, probs_ref, tok_ref,
                 vb_acc, vi_acc, vn_acc, *, V, C, nblk):
    r = pl.program_id(0)
    c = pl.program_id(1)
    neginf = jnp.float32(-jnp.inf)
    big = jnp.int32(V)

    @pl.when(c == 0)
    def _init():
        vb_acc[...] = jnp.full((SUB, SUBW), neginf, jnp.float32)
        vi_acc[...] = jnp.zeros((SUB, SUBW), jnp.int32)
        vn_acc[...] = jnp.full((SUB, SUBW), big, jnp.int32)

    t_raw = temps_ref[r]
    t = jnp.where(t_raw < 1e-5, jnp.float32(1.0), t_raw)
    rt = jnp.float32(1.0) / t
    m = m_ref[r]
    rs = jnp.float32(1.0) / s_ref[r]

    sub = jax.lax.broadcasted_iota(jnp.int32, (SUB, SUBW), 0)
    lane = jax.lax.broadcasted_iota(jnp.int32, (SUB, SUBW), 1)
    base_flat = sub * C + lane

    vb = vb_acc[...]
    vi = vi_acc[...]
    vn = vn_acc[...]
    for k in range(WB // SUBW):
        x = logits_ref[0, :, k * SUBW:(k + 1) * SUBW]
        e = jnp.exp(x * rt - m)
        probs = e * rs
        probs_ref[0, :, k * SUBW:(k + 1) * SUBW] = probs

        col0 = c * WB + k * SUBW
        valid = (col0 + lane) < C
        flat = base_flat + col0

        j = (r * V + flat).astype(jnp.uint32)
        bits = _threefry_bits(j)
        uf = jax.lax.bitcast_convert_type(
            (bits >> jnp.uint32(9)) | jnp.uint32(0x3F800000), jnp.float32
        ) - jnp.float32(1.0)
        q = -jnp.log1p(-uf)
        ratio = probs / q

        ok = valid & (ratio == ratio)
        r2 = jnp.where(ok, ratio, neginf)
        upd = r2 > vb
        vb = jnp.maximum(vb, r2)
        vi = jnp.where(upd, flat, vi)
        vn = jnp.minimum(vn, jnp.where(valid & (ratio != ratio), flat, big))
    vb_acc[...] = vb
    vi_acc[...] = vi
    vn_acc[...] = vn

    @pl.when(c == nblk - 1)
    def _emit():
        mx = jnp.max(vb)
        bidx = jnp.min(jnp.where(vb == mx, vi, big))
        nidx = jnp.min(vn)
        sampled = jnp.where(nidx < big, nidx, bidx)
        tok = jnp.where(t_raw < 1e-5, g_ref[r], sampled)
        tok_ref[...] = jnp.reshape(tok, (1, 1, 1))


def kernel(logits, temperatures):
    B, V = logits.shape
    C = V // SUB
    x3 = logits.reshape(B, SUB, C)

    nblk_a = pl.cdiv(C, WA)
    m3, g3, s3 = pl.pallas_call(
        functools.partial(_stats_body, C=C, nblk=nblk_a),
        grid=(B, nblk_a),
        in_specs=[
            pl.BlockSpec(memory_space=pltpu.SMEM),
            pl.BlockSpec((1, SUB, WA), lambda r, c: (r, 0, c)),
        ],
        out_specs=[
            pl.BlockSpec((1, 1, 1), lambda r, c: (r, 0, 0)),
            pl.BlockSpec((1, 1, 1), lambda r, c: (r, 0, 0)),
            pl.BlockSpec((1, 1, 1), lambda r, c: (r, 0, 0)),
        ],
        out_shape=[
            jax.ShapeDtypeStruct((B, 1, 1), jnp.float32),
            jax.ShapeDtypeStruct((B, 1, 1), jnp.int32),
            jax.ShapeDtypeStruct((B, 1, 1), jnp.float32),
        ],
        scratch_shapes=[
            pltpu.VMEM((SUB, SUBW), jnp.float32),
            pltpu.VMEM((SUB, SUBW), jnp.int32),
            pltpu.VMEM((SUB, SUBW), jnp.float32),
        ],
    )(temperatures, x3)

    nblk_b = pl.cdiv(C, WB)
    probs3, tok3 = pl.pallas_call(
        functools.partial(_sample_body, V=V, C=C, nblk=nblk_b),
        grid=(B, nblk_b),
        in_specs=[
            pl.BlockSpec(memory_space=pltpu.SMEM),
            pl.BlockSpec(memory_space=pltpu.SMEM),
            pl.BlockSpec(memory_space=pltpu.SMEM),
            pl.BlockSpec(memory_space=pltpu.SMEM),
            pl.BlockSpec((1, SUB, WB), lambda r, c: (r, 0, c)),
        ],
        out_specs=[
            pl.BlockSpec((1, SUB, WB), lambda r, c: (r, 0, c)),
            pl.BlockSpec((1, 1, 1), lambda r, c: (r, 0, 0)),
        ],
        out_shape=[
            jax.ShapeDtypeStruct((B, SUB, C), jnp.float32),
            jax.ShapeDtypeStruct((B, 1, 1), jnp.int32),
        ],
        scratch_shapes=[
            pltpu.VMEM((SUB, SUBW), jnp.float32),
            pltpu.VMEM((SUB, SUBW), jnp.int32),
            pltpu.VMEM((SUB, SUBW), jnp.int32),
        ],
    )(temperatures, m3.reshape(B), s3.reshape(B), g3.reshape(B), x3)

    return tok3.reshape(B), probs3.reshape(B, V)


# speedup vs baseline: 6.8940x; 1.3012x over previous
import functools

import jax
import jax.numpy as jnp
from jax.experimental import pallas as pl
from jax.experimental.pallas import tpu as pltpu

SUB = 8
SUBW = 512


def _rotl(x, d):
    return (x << jnp.uint32(d)) | (x >> jnp.uint32(32 - d))


def _threefry_bits(j):
    ks0 = jnp.uint32(0)
    ks1 = jnp.uint32(1)
    ks2 = jnp.uint32(0x1BD11BDA) ^ ks0 ^ ks1
    ks = (ks0, ks1, ks2)
    rotations = ((13, 15, 26, 6), (17, 29, 16, 24))
    x0 = jnp.zeros_like(j) + ks0
    x1 = j + ks1
    for i in range(5):
        for r in rotations[i % 2]:
            x0 = x0 + x1
            x1 = _rotl(x1, r)
            x1 = x1 ^ x0
        x0 = x0 + ks[(i + 1) % 3]
        x1 = x1 + ks[(i + 2) % 3] + jnp.uint32(i + 1)
    return x0 ^ x1


def _iotas(shape, C):
    sub = jax.lax.broadcasted_iota(jnp.int32, shape, 0)
    lane = jax.lax.broadcasted_iota(jnp.int32, shape, 1)
    return sub * C + lane


def _row_body(temps_ref, logits_ref, probs_ref, tok_ref, *, V, C):
    r = pl.program_id(0)
    t_raw = temps_ref[r]
    t = jnp.where(t_raw < 1e-5, jnp.float32(1.0), t_raw)
    rt = jnp.float32(1.0) / t
    neginf = jnp.float32(-jnp.inf)
    big = jnp.int32(V)

    nfull = C // SUBW
    tailw = C - nfull * SUBW
    base_flat = _iotas((SUB, SUBW), C)
    tail_flat = _iotas((SUB, tailw), C) + nfull * SUBW if tailw else None

    vm = jnp.full((SUB, SUBW), neginf, jnp.float32)
    vg = jnp.full((SUB, SUBW), big, jnp.int32)
    for k in range(nfull):
        sc = logits_ref[0, :, k * SUBW:(k + 1) * SUBW] * rt
        upd = sc > vm
        vm = jnp.maximum(vm, sc)
        vg = jnp.where(upd, base_flat + k * SUBW, vg)
    m = jnp.max(vm)
    g = jnp.min(jnp.where(vm == m, vg, big))
    if tailw:
        sc = logits_ref[0, :, nfull * SUBW:C] * rt
        mt = jnp.max(sc)
        gt = jnp.min(jnp.where(sc == mt, tail_flat, big))
        g = jnp.where(mt > m, gt, g)
        m = jnp.maximum(m, mt)

    vs = jnp.zeros((SUB, SUBW), jnp.float32)
    for k in range(nfull):
        vs = vs + jnp.exp(logits_ref[0, :, k * SUBW:(k + 1) * SUBW] * rt - m)
    s = jnp.sum(vs)
    if tailw:
        s = s + jnp.sum(jnp.exp(logits_ref[0, :, nfull * SUBW:C] * rt - m))
    rs = jnp.float32(1.0) / s

    def chunk_ratio(x, flat0):
        e = jnp.exp(x * rt - m)
        probs = e * rs
        j = flat0.astype(jnp.uint32)
        bits = _threefry_bits(j)
        uf = jax.lax.bitcast_convert_type(
            (bits >> jnp.uint32(9)) | jnp.uint32(0x3F800000), jnp.float32
        ) - jnp.float32(1.0)
        q = -jnp.log1p(-uf)
        ratio = probs / q
        return probs, ratio

    rbase = r * V
    vb = jnp.full((SUB, SUBW), neginf, jnp.float32)
    vi = jnp.zeros((SUB, SUBW), jnp.int32)
    vn = jnp.full((SUB, SUBW), big, jnp.int32)
    for k in range(nfull):
        x = logits_ref[0, :, k * SUBW:(k + 1) * SUBW]
        flat = base_flat + k * SUBW
        probs, ratio = chunk_ratio(x, flat + rbase)
        probs_ref[0, :, k * SUBW:(k + 1) * SUBW] = probs
        ok = ratio == ratio
        r2 = jnp.where(ok, ratio, neginf)
        upd = r2 > vb
        vb = jnp.maximum(vb, r2)
        vi = jnp.where(upd, flat, vi)
        vn = jnp.minimum(vn, jnp.where(ok, big, flat))
    mx = jnp.max(vb)
    bidx = jnp.min(jnp.where(vb == mx, vi, big))
    nidx = jnp.min(vn)
    if tailw:
        x = logits_ref[0, :, nfull * SUBW:C]
        flat = tail_flat
        probs, ratio = chunk_ratio(x, flat + rbase)
        probs_ref[0, :, nfull * SUBW:C] = probs
        ok = ratio == ratio
        r2 = jnp.where(ok, ratio, neginf)
        mxt = jnp.max(r2)
        bt = jnp.min(jnp.where(r2 == mxt, flat, big))
        nt = jnp.min(jnp.where(ok, big, flat))
        bidx = jnp.where(mxt > mx, bt, bidx)
        mx = jnp.maximum(mx, mxt)
        nidx = jnp.minimum(nidx, nt)

    sampled = jnp.where(nidx < big, nidx, bidx)
    tok = jnp.where(t_raw < 1e-5, g, sampled)
    tok_ref[...] = jnp.reshape(tok, (1, 1, 1))


def kernel(logits, temperatures):
    B, V = logits.shape
    C = V // SUB
    x3 = logits.reshape(B, SUB, C)
    probs3, tok3 = pl.pallas_call(
        functools.partial(_row_body, V=V, C=C),
        grid=(B,),
        in_specs=[
            pl.BlockSpec(memory_space=pltpu.SMEM),
            pl.BlockSpec((1, SUB, C), lambda r: (r, 0, 0)),
        ],
        out_specs=[
            pl.BlockSpec((1, SUB, C), lambda r: (r, 0, 0)),
            pl.BlockSpec((1, 1, 1), lambda r: (r, 0, 0)),
        ],
        out_shape=[
            jax.ShapeDtypeStruct((B, SUB, C), jnp.float32),
            jax.ShapeDtypeStruct((B, 1, 1), jnp.int32),
        ],
    )(temperatures, x3)
    return tok3.reshape(B), probs3.reshape(B, V)
